# baseline (speedup 1.0000x reference)
import functools
import jax
import jax.numpy as jnp
import numpy as np
from jax import lax
from jax.experimental import pallas as pl
from jax.experimental.pallas import tpu as pltpu
from jax.experimental.pallas import tpu_sc as plsc

D_K = 1024
S_K = 8192
B_K = 4
SCALE_K = float(np.sqrt(D_K))
NC_K, NS_K, L_K = 2, 16, 16
NW_K = NC_K * NS_K
ROWS_W = S_K // NW_K
R_K = 32
CHUNKS_K = ROWS_W // R_K
CELEMS = R_K * D_K
GROUPS_K = CELEMS // L_K

_mesh = plsc.VectorSubcoreMesh(core_axis_name="c", subcore_axis_name="s")


@functools.partial(
    pl.kernel,
    mesh=_mesh,
    out_type=jax.ShapeDtypeStruct((B_K * S_K * D_K,), jnp.float32),
    scratch_types=[
        pltpu.VMEM((CELEMS,), jnp.float32),
        pltpu.VMEM((CELEMS,), jnp.float32),
    ],
)
def _sc_add(x_hbm, pe_hbm, out_hbm, pe_buf, x_buf):
    wid = lax.axis_index("s") * NC_K + lax.axis_index("c")
    base = wid * ROWS_W * D_K

    def chunk_body(c, carry):
        off = pl.multiple_of(base + c * CELEMS, CELEMS)
        pltpu.sync_copy(pe_hbm.at[pl.ds(off, CELEMS)], pe_buf)

        def scale_body(g, c2):
            sl = pl.ds(g * L_K, L_K)
            pe_buf[sl] = pe_buf[sl] * SCALE_K
            return c2

        lax.fori_loop(0, GROUPS_K, scale_body, 0)

        for b in range(B_K):
            xoff = pl.multiple_of(b * S_K * D_K + off, CELEMS)
            pltpu.sync_copy(x_hbm.at[pl.ds(xoff, CELEMS)], x_buf)

            def add_body(g, c2):
                sl = pl.ds(g * L_K, L_K)
                x_buf[sl] = x_buf[sl] + pe_buf[sl]
                return c2

            lax.fori_loop(0, GROUPS_K, add_body, 0)
            pltpu.sync_copy(x_buf, out_hbm.at[pl.ds(xoff, CELEMS)])
        return carry

    lax.fori_loop(0, CHUNKS_K, chunk_body, 0)


def kernel(x, pe_weight):
    b, s, d = x.shape
    out = _sc_add(x.reshape(-1), pe_weight[:s].reshape(-1))
    return out.reshape(b, s, d)


# speedup vs baseline: 2.0032x; 2.0032x over previous
import functools
import jax
import jax.numpy as jnp
import numpy as np
from jax import lax
from jax.experimental import pallas as pl
from jax.experimental.pallas import tpu as pltpu
from jax.experimental.pallas import tpu_sc as plsc

D_K = 1024
S_K = 8192
B_K = 4
SCALE_K = float(np.sqrt(D_K))
NC_K, NS_K, L_K = 2, 16, 16
NW_K = NC_K * NS_K
ROWS_W = S_K // NW_K
R_K = 16
CHUNKS_K = ROWS_W // R_K
CELEMS = R_K * D_K
NB_K = 5
LOOK_K = 2
NT_K = CHUNKS_K * B_K

_mesh = plsc.VectorSubcoreMesh(core_axis_name="c", subcore_axis_name="s")


@functools.partial(
    pl.kernel,
    mesh=_mesh,
    out_type=jax.ShapeDtypeStruct((B_K * S_K * D_K,), jnp.float32),
    scratch_types=(
        [pltpu.VMEM((CELEMS,), jnp.float32) for _ in range(NB_K)]
        + [pltpu.VMEM((CELEMS,), jnp.float32) for _ in range(2)]
        + [pltpu.SemaphoreType.DMA for _ in range(NB_K)]
        + [pltpu.SemaphoreType.DMA for _ in range(NB_K)]
        + [pltpu.SemaphoreType.DMA for _ in range(2)]
    ),
)
def _sc_add(x_hbm, pe_hbm, out_hbm, xb0, xb1, xb2, xb3, xb4, pb0, pb1,
            si0, si1, si2, si3, si4, so0, so1, so2, so3, so4, sp0, sp1):
    xb = [xb0, xb1, xb2, xb3, xb4]
    si = [si0, si1, si2, si3, si4]
    so = [so0, so1, so2, so3, so4]
    pb = [pb0, pb1]
    sp = [sp0, sp1]

    wid = lax.axis_index("s") * NC_K + lax.axis_index("c")
    base = wid * ROWS_W * D_K

    def pe_off(c):
        return pl.multiple_of(base + c * CELEMS, CELEMS)

    def x_off(t):
        c, b = divmod(t, B_K)
        return pl.multiple_of(b * S_K * D_K + base + c * CELEMS, CELEMS)

    def in_copy(t):
        return pltpu.make_async_copy(
            x_hbm.at[pl.ds(x_off(t), CELEMS)], xb[t % NB_K], si[t % NB_K])

    def out_copy(t):
        return pltpu.make_async_copy(
            xb[t % NB_K], out_hbm.at[pl.ds(x_off(t), CELEMS)], so[t % NB_K])

    def pe_copy(c):
        return pltpu.make_async_copy(
            pe_hbm.at[pl.ds(pe_off(c), CELEMS)], pb[c % 2], sp[c % 2])

    pe_copy(0).start()
    pe_copy(1).start()
    for t in range(LOOK_K):
        in_copy(t).start()

    for t in range(NT_K):
        nxt = t + LOOK_K
        if nxt < NT_K:
            if nxt >= NB_K:
                out_copy(nxt - NB_K).wait()
            in_copy(nxt).start()

        c, b = divmod(t, B_K)
        if b == 0:
            pe_copy(c).wait()
            peb = pb[c % 2]

            @plsc.parallel_loop(0, CELEMS, L_K, unroll=8)
            def _scale(i):
                peb[pl.ds(i, L_K)] = peb[pl.ds(i, L_K)] * SCALE_K

        in_copy(t).wait()
        xbt = xb[t % NB_K]
        pebt = pb[c % 2]

        @plsc.parallel_loop(0, CELEMS, L_K, unroll=8)
        def _add(i):
            xbt[pl.ds(i, L_K)] = xbt[pl.ds(i, L_K)] + pebt[pl.ds(i, L_K)]

        out_copy(t).start()

        if b == B_K - 1 and c + 2 < CHUNKS_K:
            pe_copy(c + 2).start()

    for t in range(NT_K - NB_K, NT_K):
        out_copy(t).wait()


def kernel(x, pe_weight):
    b, s, d = x.shape
    out = _sc_add(x.reshape(-1), pe_weight[:s].reshape(-1))
    return out.reshape(b, s, d)


# speedup vs baseline: 5.9150x; 2.9529x over previous
import functools
import jax
import jax.numpy as jnp
import numpy as np
from jax import lax
from jax.experimental import pallas as pl
from jax.experimental.pallas import tpu as pltpu
from jax.experimental.pallas import tpu_sc as plsc

D_K = 1024
S_K = 8192
B_K = 4
SCALE_K = float(np.sqrt(D_K))
NC_K, NS_K, L_K = 2, 16, 16
NW_K = NC_K * NS_K
ROWS_W = S_K // NW_K
R_K = 16
CHUNKS_K = ROWS_W // R_K
GROUPS_K = R_K * D_K // L_K
CPR_K = D_K // L_K
NB_K = 5
LOOK_K = 2
NT_K = CHUNKS_K * B_K

_mesh = plsc.VectorSubcoreMesh(core_axis_name="c", subcore_axis_name="s")


@functools.partial(
    pl.kernel,
    mesh=_mesh,
    out_type=jax.ShapeDtypeStruct((B_K * S_K, D_K), jnp.float32),
    scratch_types=(
        [pltpu.VMEM((R_K, D_K), jnp.float32) for _ in range(NB_K + 2)]
        + [pltpu.SemaphoreType.DMA for _ in range(2 * NB_K + 2)]
    ),
)
def _sc_add(x_hbm, pe_hbm, out_hbm, xb0, xb1, xb2, xb3, xb4, pb0, pb1,
            si0, si1, si2, si3, si4, so0, so1, so2, so3, so4, sp0, sp1):
    xb = [xb0, xb1, xb2, xb3, xb4]
    si = [si0, si1, si2, si3, si4]
    so = [so0, so1, so2, so3, so4]
    pb = [pb0, pb1]
    sp = [sp0, sp1]

    wid = lax.axis_index("s") * NC_K + lax.axis_index("c")
    base = wid * ROWS_W

    def pe_row(c):
        return pl.multiple_of(base + c * R_K, R_K)

    def x_row(t):
        c, b = divmod(t, B_K)
        return pl.multiple_of(b * S_K + base + c * R_K, R_K)

    def in_copy(t):
        return pltpu.make_async_copy(
            x_hbm.at[pl.ds(x_row(t), R_K)], xb[t % NB_K], si[t % NB_K])

    def out_copy(t):
        return pltpu.make_async_copy(
            xb[t % NB_K], out_hbm.at[pl.ds(x_row(t), R_K)], so[t % NB_K])

    def pe_copy(c):
        return pltpu.make_async_copy(
            pe_hbm.at[pl.ds(pe_row(c), R_K)], pb[c % 2], sp[c % 2])

    pe_copy(0).start()
    pe_copy(1).start()
    for t in range(LOOK_K):
        in_copy(t).start()

    for t in range(NT_K):
        nxt = t + LOOK_K
        if nxt < NT_K:
            if nxt >= NB_K:
                out_copy(nxt - NB_K).wait()
            in_copy(nxt).start()

        c, b = divmod(t, B_K)
        if b == 0:
            pe_copy(c).wait()
            peb = pb[c % 2]

            @plsc.parallel_loop(0, GROUPS_K, 1, unroll=8)
            def _scale(g):
                r = g >> 6
                col = (g & (CPR_K - 1)) * L_K
                sl = pl.ds(col, L_K)
                peb[r, sl] = peb[r, sl] * SCALE_K

        in_copy(t).wait()
        xbt = xb[t % NB_K]
        pebt = pb[c % 2]

        @plsc.parallel_loop(0, GROUPS_K, 1, unroll=8)
        def _add(g):
            r = g >> 6
            col = (g & (CPR_K - 1)) * L_K
            sl = pl.ds(col, L_K)
            xbt[r, sl] = xbt[r, sl] + pebt[r, sl]

        out_copy(t).start()

        if b == B_K - 1 and c + 2 < CHUNKS_K:
            pe_copy(c + 2).start()

    for t in range(NT_K - NB_K, NT_K):
        out_copy(t).wait()


def kernel(x, pe_weight):
    b, s, d = x.shape
    out = _sc_add(x.reshape(b * s, d), pe_weight[:s])
    return out.reshape(b, s, d)
